# baseline (speedup 1.0000x reference)
import functools

import jax
import jax.numpy as jnp
from jax import lax
from jax.experimental import pallas as pl
from jax.experimental.pallas import tpu as pltpu
from jax.experimental.pallas import tpu_sc as plsc

N = 10000
M = 32
ATOM = 128
NBR = 16
GROUPS = 8
E = N * M
ROWS = E // GROUPS
NW = 32
NPAD = 10240


BR = 2048
GRID_A = -(-ROWS // BR)
BN = BR * GROUPS // M
SROWS = GRID_A * BR * GROUPS // 128
E_SC = SROWS * 128
TRASH = N


def _edge_scalar_body(r_ref, idx_ref, w1b_ref, b1t_ref, k2_ref, b2s_ref,
                      s_ref, oidx_ref):
    x = r_ref[...]
    h = jnp.dot(x, w1b_ref[...], preferred_element_type=jnp.float32,
                precision=lax.Precision.HIGHEST)
    h = h + b1t_ref[...]
    h = jnp.maximum(h, 0.0) + jnp.log1p(jnp.exp(-jnp.abs(h)))
    s8 = jnp.dot(h, k2_ref[...], preferred_element_type=jnp.float32,
                 precision=lax.Precision.HIGHEST)
    s8 = s8 + b2s_ref[0]

    eye = (lax.broadcasted_iota(jnp.int32, (128, 128), 0)
           == lax.broadcasted_iota(jnp.int32, (128, 128), 1)
           ).astype(jnp.float32)

    s3 = s8.reshape(BR // 16, 16, GROUPS)
    s2d = sum(
        jnp.dot(s3[:, k, :], eye[8 * k:8 * (k + 1), :],
                preferred_element_type=jnp.float32,
                precision=lax.Precision.HIGHEST)
        for k in range(16))
    s_ref[...] = s2d

    idxf = idx_ref[...].astype(jnp.float32)
    i3 = idxf.reshape(BN // 4, 4, M)
    idf = sum(
        jnp.dot(i3[:, k, :], eye[32 * k:32 * (k + 1), :],
                preferred_element_type=jnp.float32,
                precision=lax.Precision.HIGHEST)
        for k in range(4))
    g = pl.program_id(0)
    br_out = BR // 16
    row = g * br_out + lax.broadcasted_iota(jnp.int32, (br_out, 128), 0)
    oidx_ref[...] = jnp.where(row < E // 128, idf.astype(jnp.int32), TRASH)


def _edge_scalars(r8, nbr_idx, w1b, b1t, k2, b2s):
    return pl.pallas_call(
        _edge_scalar_body,
        grid=(GRID_A,),
        in_specs=[
            pl.BlockSpec((BR, 128), lambda g: (g, 0)),
            pl.BlockSpec((BN, M), lambda g: (g, 0)),
            pl.BlockSpec((128, 128), lambda g: (0, 0)),
            pl.BlockSpec((1, 128), lambda g: (0, 0)),
            pl.BlockSpec((128, GROUPS), lambda g: (0, 0)),
            pl.BlockSpec(memory_space=pltpu.SMEM),
        ],
        out_specs=[
            pl.BlockSpec((BR // 16, 128), lambda g: (g, 0)),
            pl.BlockSpec((BR // 16, 128), lambda g: (g, 0)),
        ],
        out_shape=[
            jax.ShapeDtypeStruct((SROWS, 128), jnp.float32),
            jax.ShapeDtypeStruct((SROWS, 128), jnp.int32),
        ],
    )(r8, nbr_idx, w1b, b1t, k2, b2s)


E_PER_W = E_SC // NW
VECS_PER_W = E_PER_W // 16


def _sc_scatter(flat_idx, s_flat):
    mesh = plsc.VectorSubcoreMesh(core_axis_name="c", subcore_axis_name="s")

    @functools.partial(
        pl.kernel,
        mesh=mesh,
        out_type=[
            jax.ShapeDtypeStruct((NW, NPAD), jnp.float32),
            jax.ShapeDtypeStruct((NW, NPAD), jnp.float32),
        ],
        scratch_types=[
            pltpu.VMEM((E_PER_W,), jnp.int32),
            pltpu.VMEM((E_PER_W,), jnp.float32),
            pltpu.VMEM((NPAD,), jnp.float32),
            pltpu.VMEM((NPAD,), jnp.float32),
        ],
        compiler_params=pltpu.CompilerParams(needs_layout_passes=False),
    )
    def scatter_kernel(idx_hbm, s_hbm, osum_hbm, ocnt_hbm,
                       idx_v, s_v, sum_v, cnt_v):
        wid = lax.axis_index("s") * 2 + lax.axis_index("c")
        base = wid * E_PER_W
        pltpu.sync_copy(idx_hbm.at[pl.ds(base, E_PER_W)], idx_v)
        pltpu.sync_copy(s_hbm.at[pl.ds(base, E_PER_W)], s_v)

        zeros = jnp.zeros((16,), jnp.float32)

        def zero_body(i, _):
            sum_v[pl.ds(i * 16, 16)] = zeros
            cnt_v[pl.ds(i * 16, 16)] = zeros
            return ()

        lax.fori_loop(0, NPAD // 16, zero_body, ())

        ones = jnp.ones((16,), jnp.float32)

        def acc_body(i, _):
            idx16 = idx_v[pl.ds(i * 16, 16)]
            s16 = s_v[pl.ds(i * 16, 16)]
            plsc.addupdate_scatter(sum_v, [idx16], s16)
            plsc.addupdate_scatter(cnt_v, [idx16], ones)
            return ()

        lax.fori_loop(0, VECS_PER_W, acc_body, ())

        pltpu.sync_copy(sum_v, osum_hbm.at[wid])
        pltpu.sync_copy(cnt_v, ocnt_hbm.at[wid])

    return scatter_kernel(flat_idx, s_flat)


def _finish_body(sum_ref, cnt_ref, atom_ref, tpw_ref, out_ref):
    ssum = jnp.sum(sum_ref[...], axis=0)
    cnt = jnp.sum(cnt_ref[...], axis=0)
    mean = ssum / jnp.maximum(cnt, 1.0)
    scale = mean * (1.0 / jnp.sqrt(float(ATOM)))
    p = jnp.dot(atom_ref[...], tpw_ref[...], preferred_element_type=jnp.float32)
    out_ref[...] = p * scale[:, None]


def _finish(psum, pcnt, atom_fea, tp_w):
    BC = 1024
    grid = NPAD // BC
    return pl.pallas_call(
        _finish_body,
        grid=(grid,),
        in_specs=[
            pl.BlockSpec((NW, BC), lambda g: (0, g)),
            pl.BlockSpec((NW, BC), lambda g: (0, g)),
            pl.BlockSpec((BC, ATOM), lambda g: (g, 0)),
            pl.BlockSpec((ATOM, ATOM), lambda g: (0, 0)),
        ],
        out_specs=pl.BlockSpec((BC, ATOM), lambda g: (g, 0)),
        out_shape=jax.ShapeDtypeStruct((N, ATOM), jnp.float32),
    )(psum, pcnt, atom_fea, tp_w)


def kernel(atom_fea, nbr_fea, nbr_idx, pos, W1, b1, W2, b2, tp_w):
    del pos
    w1b = jnp.kron(jnp.eye(GROUPS, dtype=jnp.float32), W1)
    b1t = jnp.tile(b1, GROUPS)[None, :]
    k2 = jnp.kron(jnp.eye(GROUPS, dtype=jnp.float32), W2[:, 0:1])
    b2s = jnp.full((1,), b2[0], dtype=jnp.float32)

    r8 = nbr_fea.reshape(ROWS, 128)
    s2d, idx2d = _edge_scalars(r8, nbr_idx.astype(jnp.int32),
                               w1b, b1t, k2, b2s)
    s_flat = s2d.reshape(E_SC)
    flat_idx = idx2d.reshape(E_SC)
    psum, pcnt = _sc_scatter(flat_idx, s_flat)

    return _finish(psum, pcnt, atom_fea, tp_w)


# speedup vs baseline: 1.2065x; 1.2065x over previous
import functools

import jax
import jax.numpy as jnp
from jax import lax
from jax.experimental import pallas as pl
from jax.experimental.pallas import tpu as pltpu
from jax.experimental.pallas import tpu_sc as plsc

N = 10000
M = 32
ATOM = 128
NBR = 16
GROUPS = 8
E = N * M
ROWS = E // GROUPS
NW = 32
NPAD = 10240


BR = 2048
GRID_A = -(-ROWS // BR)
BN = BR * GROUPS // M
SROWS = GRID_A * BR * GROUPS // 128
E_SC = SROWS * 128
TRASH = N


def _edge_scalar_body(r_ref, idx_ref, w1b_ref, b1t_ref, k2_ref, b2s_ref,
                      s_ref, oidx_ref):
    x = r_ref[...]
    h = jnp.dot(x, w1b_ref[...], preferred_element_type=jnp.float32)
    h = h + b1t_ref[...]
    h = jnp.maximum(h, 0.0) + jnp.log1p(jnp.exp(-jnp.abs(h)))
    s8 = jnp.dot(h, k2_ref[...], preferred_element_type=jnp.float32)
    s8 = s8 + b2s_ref[0]

    eye = (lax.broadcasted_iota(jnp.int32, (128, 128), 0)
           == lax.broadcasted_iota(jnp.int32, (128, 128), 1)
           ).astype(jnp.float32)

    s3 = s8.reshape(BR // 16, 16, GROUPS)
    s2d = sum(
        jnp.dot(s3[:, k, :], eye[8 * k:8 * (k + 1), :],
                preferred_element_type=jnp.float32,
                precision=lax.Precision.HIGHEST)
        for k in range(16))
    s_ref[...] = s2d

    idxf = idx_ref[...].astype(jnp.float32)
    i3 = idxf.reshape(BN // 4, 4, M)
    idf = sum(
        jnp.dot(i3[:, k, :], eye[32 * k:32 * (k + 1), :],
                preferred_element_type=jnp.float32,
                precision=lax.Precision.HIGHEST)
        for k in range(4))
    g = pl.program_id(0)
    br_out = BR // 16
    row = g * br_out + lax.broadcasted_iota(jnp.int32, (br_out, 128), 0)
    oidx_ref[...] = jnp.where(row < E // 128, idf.astype(jnp.int32), TRASH)


def _edge_scalars(r8, nbr_idx, w1b, b1t, k2, b2s):
    return pl.pallas_call(
        _edge_scalar_body,
        grid=(GRID_A,),
        in_specs=[
            pl.BlockSpec((BR, 128), lambda g: (g, 0)),
            pl.BlockSpec((BN, M), lambda g: (g, 0)),
            pl.BlockSpec((128, 128), lambda g: (0, 0)),
            pl.BlockSpec((1, 128), lambda g: (0, 0)),
            pl.BlockSpec((128, GROUPS), lambda g: (0, 0)),
            pl.BlockSpec(memory_space=pltpu.SMEM),
        ],
        out_specs=[
            pl.BlockSpec((BR // 16, 128), lambda g: (g, 0)),
            pl.BlockSpec((BR // 16, 128), lambda g: (g, 0)),
        ],
        out_shape=[
            jax.ShapeDtypeStruct((SROWS, 128), jnp.float32),
            jax.ShapeDtypeStruct((SROWS, 128), jnp.int32),
        ],
    )(r8, nbr_idx, w1b, b1t, k2, b2s)


E_PER_W = E_SC // NW
VECS_PER_W = E_PER_W // 16


def _sc_scatter(flat_idx, s_flat):
    mesh = plsc.VectorSubcoreMesh(core_axis_name="c", subcore_axis_name="s")

    @functools.partial(
        pl.kernel,
        mesh=mesh,
        out_type=[
            jax.ShapeDtypeStruct((NW, NPAD), jnp.float32),
            jax.ShapeDtypeStruct((NW, NPAD), jnp.float32),
        ],
        scratch_types=[
            pltpu.VMEM((E_PER_W,), jnp.int32),
            pltpu.VMEM((E_PER_W,), jnp.float32),
            pltpu.VMEM((NPAD,), jnp.float32),
            pltpu.VMEM((NPAD,), jnp.float32),
        ],
        compiler_params=pltpu.CompilerParams(needs_layout_passes=False),
    )
    def scatter_kernel(idx_hbm, s_hbm, osum_hbm, ocnt_hbm,
                       idx_v, s_v, sum_v, cnt_v):
        wid = lax.axis_index("s") * 2 + lax.axis_index("c")
        base = wid * E_PER_W
        pltpu.sync_copy(idx_hbm.at[pl.ds(base, E_PER_W)], idx_v)
        pltpu.sync_copy(s_hbm.at[pl.ds(base, E_PER_W)], s_v)

        zeros = jnp.zeros((16,), jnp.float32)

        def zero_body(i, _):
            sum_v[pl.ds(i * 16, 16)] = zeros
            cnt_v[pl.ds(i * 16, 16)] = zeros
            return ()

        lax.fori_loop(0, NPAD // 16, zero_body, ())

        ones = jnp.ones((16,), jnp.float32)

        def acc_body(i, _):
            idx16 = idx_v[pl.ds(i * 16, 16)]
            s16 = s_v[pl.ds(i * 16, 16)]
            plsc.addupdate_scatter(sum_v, [idx16], s16)
            plsc.addupdate_scatter(cnt_v, [idx16], ones)
            return ()

        lax.fori_loop(0, VECS_PER_W, acc_body, ())

        pltpu.sync_copy(sum_v, osum_hbm.at[wid])
        pltpu.sync_copy(cnt_v, ocnt_hbm.at[wid])

    return scatter_kernel(flat_idx, s_flat)


def _finish_body(sum_ref, cnt_ref, atom_ref, tpw_ref, out_ref):
    ssum = jnp.sum(sum_ref[...], axis=0)
    cnt = jnp.sum(cnt_ref[...], axis=0)
    mean = ssum / jnp.maximum(cnt, 1.0)
    scale = mean * (1.0 / jnp.sqrt(float(ATOM)))
    p = jnp.dot(atom_ref[...], tpw_ref[...], preferred_element_type=jnp.float32)
    out_ref[...] = p * scale[:, None]


def _finish(psum, pcnt, atom_fea, tp_w):
    BC = 1024
    grid = NPAD // BC
    return pl.pallas_call(
        _finish_body,
        grid=(grid,),
        in_specs=[
            pl.BlockSpec((NW, BC), lambda g: (0, g)),
            pl.BlockSpec((NW, BC), lambda g: (0, g)),
            pl.BlockSpec((BC, ATOM), lambda g: (g, 0)),
            pl.BlockSpec((ATOM, ATOM), lambda g: (0, 0)),
        ],
        out_specs=pl.BlockSpec((BC, ATOM), lambda g: (g, 0)),
        out_shape=jax.ShapeDtypeStruct((N, ATOM), jnp.float32),
    )(psum, pcnt, atom_fea, tp_w)


def kernel(atom_fea, nbr_fea, nbr_idx, pos, W1, b1, W2, b2, tp_w):
    del pos
    w1b = jnp.kron(jnp.eye(GROUPS, dtype=jnp.float32), W1)
    b1t = jnp.tile(b1, GROUPS)[None, :]
    k2 = jnp.kron(jnp.eye(GROUPS, dtype=jnp.float32), W2[:, 0:1])
    b2s = jnp.full((1,), b2[0], dtype=jnp.float32)

    r8 = nbr_fea.reshape(ROWS, 128)
    s2d, idx2d = _edge_scalars(r8, nbr_idx.astype(jnp.int32),
                               w1b, b1t, k2, b2s)
    s_flat = s2d.reshape(E_SC)
    flat_idx = idx2d.reshape(E_SC)
    psum, pcnt = _sc_scatter(flat_idx, s_flat)

    return _finish(psum, pcnt, atom_fea, tp_w)


# speedup vs baseline: 4.6670x; 3.8681x over previous
import functools

import jax
import jax.numpy as jnp
from jax import lax
from jax.experimental import pallas as pl
from jax.experimental.pallas import tpu as pltpu
from jax.experimental.pallas import tpu_sc as plsc

N = 10000
M = 32
ATOM = 128
NBR = 16
GROUPS = 8
E = N * M
ROWS = E // GROUPS
NW = 32
NPAD = 10240


BN = 2048
GRID_A = -(-N // BN)
NCOL = GRID_A * BN
E_SC = M * NCOL
TRASH = N


def _edge_scalar_body(fea_ref, idx_ref, w1t_ref, w2s_ref, b1c_ref, b2s_ref,
                      s_ref, oidx_ref):
    x = fea_ref[...].reshape(M * NBR, BN)
    rows = []
    for gp in range(4):
        xg = x[128 * gp:128 * (gp + 1), :]
        z = jnp.dot(w1t_ref[...], xg, preferred_element_type=jnp.float32)
        z = z + b1c_ref[...]
        hg = jnp.maximum(z, 0.0) + jnp.log1p(jnp.exp(-jnp.abs(z)))
        rows.append(jnp.dot(w2s_ref[...], hg,
                            preferred_element_type=jnp.float32))
    s = jnp.concatenate(rows, axis=0) + b2s_ref[0]

    g = pl.program_id(0)
    col = g * BN + lax.broadcasted_iota(jnp.int32, (M, BN), 1)
    idx2 = jnp.where(col < N, idx_ref[...], TRASH)

    s_ref[...] = jnp.stack(
        [s[:, 128 * t:128 * (t + 1)] for t in range(BN // 128)], axis=1)
    oidx_ref[...] = jnp.stack(
        [idx2[:, 128 * t:128 * (t + 1)] for t in range(BN // 128)], axis=1)


def _edge_scalars(fea_t, idx_t, w1t, w2s, b1c, b2s):
    bt = BN // 128
    return pl.pallas_call(
        _edge_scalar_body,
        grid=(GRID_A,),
        in_specs=[
            pl.BlockSpec((M, NBR, BN), lambda g: (0, 0, g)),
            pl.BlockSpec((M, BN), lambda g: (0, g)),
            pl.BlockSpec((128, 128), lambda g: (0, 0)),
            pl.BlockSpec((GROUPS, 128), lambda g: (0, 0)),
            pl.BlockSpec((128, 1), lambda g: (0, 0)),
            pl.BlockSpec(memory_space=pltpu.SMEM),
        ],
        out_specs=[
            pl.BlockSpec((M, bt, 128), lambda g: (0, g, 0)),
            pl.BlockSpec((M, bt, 128), lambda g: (0, g, 0)),
        ],
        out_shape=[
            jax.ShapeDtypeStruct((M, NCOL // 128, 128), jnp.float32),
            jax.ShapeDtypeStruct((M, NCOL // 128, 128), jnp.int32),
        ],
    )(fea_t, idx_t, w1t, w2s, b1c, b2s)


E_PER_W = E_SC // NW
VECS_PER_W = E_PER_W // 16


def _sc_scatter(flat_idx, s_flat):
    mesh = plsc.VectorSubcoreMesh(core_axis_name="c", subcore_axis_name="s")

    @functools.partial(
        pl.kernel,
        mesh=mesh,
        out_type=[
            jax.ShapeDtypeStruct((NW, NPAD), jnp.float32),
            jax.ShapeDtypeStruct((NW, NPAD), jnp.float32),
        ],
        scratch_types=[
            pltpu.VMEM((E_PER_W,), jnp.int32),
            pltpu.VMEM((E_PER_W,), jnp.float32),
            pltpu.VMEM((NPAD,), jnp.float32),
            pltpu.VMEM((NPAD,), jnp.float32),
        ],
        compiler_params=pltpu.CompilerParams(needs_layout_passes=False),
    )
    def scatter_kernel(idx_hbm, s_hbm, osum_hbm, ocnt_hbm,
                       idx_v, s_v, sum_v, cnt_v):
        wid = lax.axis_index("s") * 2 + lax.axis_index("c")
        base = wid * E_PER_W
        pltpu.sync_copy(idx_hbm.at[pl.ds(base, E_PER_W)], idx_v)
        pltpu.sync_copy(s_hbm.at[pl.ds(base, E_PER_W)], s_v)

        zeros = jnp.zeros((16,), jnp.float32)

        def zero_body(i, _):
            sum_v[pl.ds(i * 16, 16)] = zeros
            cnt_v[pl.ds(i * 16, 16)] = zeros
            return ()

        lax.fori_loop(0, NPAD // 16, zero_body, ())

        ones = jnp.ones((16,), jnp.float32)

        def acc_body(i, _):
            idx16 = idx_v[pl.ds(i * 16, 16)]
            s16 = s_v[pl.ds(i * 16, 16)]
            plsc.addupdate_scatter(sum_v, [idx16], s16)
            plsc.addupdate_scatter(cnt_v, [idx16], ones)
            return ()

        lax.fori_loop(0, VECS_PER_W, acc_body, ())

        pltpu.sync_copy(sum_v, osum_hbm.at[wid])
        pltpu.sync_copy(cnt_v, ocnt_hbm.at[wid])

    return scatter_kernel(flat_idx, s_flat)


def _finish_body(sum_ref, cnt_ref, atom_ref, tpw_ref, out_ref):
    ssum = jnp.sum(sum_ref[...], axis=0)
    cnt = jnp.sum(cnt_ref[...], axis=0)
    mean = ssum / jnp.maximum(cnt, 1.0)
    scale = mean * (1.0 / jnp.sqrt(float(ATOM)))
    p = jnp.dot(atom_ref[...], tpw_ref[...], preferred_element_type=jnp.float32)
    out_ref[...] = p * scale[:, None]


def _finish(psum, pcnt, atom_fea, tp_w):
    BC = 1024
    grid = NPAD // BC
    return pl.pallas_call(
        _finish_body,
        grid=(grid,),
        in_specs=[
            pl.BlockSpec((NW, BC), lambda g: (0, g)),
            pl.BlockSpec((NW, BC), lambda g: (0, g)),
            pl.BlockSpec((BC, ATOM), lambda g: (g, 0)),
            pl.BlockSpec((ATOM, ATOM), lambda g: (0, 0)),
        ],
        out_specs=pl.BlockSpec((BC, ATOM), lambda g: (g, 0)),
        out_shape=jax.ShapeDtypeStruct((N, ATOM), jnp.float32),
    )(psum, pcnt, atom_fea, tp_w)


def kernel(atom_fea, nbr_fea, nbr_idx, pos, W1, b1, W2, b2, tp_w):
    del pos
    w1t = jnp.kron(jnp.eye(GROUPS, dtype=jnp.float32), W1.T)
    w2s = jnp.kron(jnp.eye(GROUPS, dtype=jnp.float32),
                   W2[:, 0][None, :])
    b1c = jnp.tile(b1, GROUPS)[:, None]
    b2s = jnp.full((1,), b2[0], dtype=jnp.float32)

    fea_t = jnp.transpose(nbr_fea, (1, 2, 0))
    idx_t = jnp.transpose(nbr_idx.astype(jnp.int32), (1, 0))

    s3d, idx3d = _edge_scalars(fea_t, idx_t, w1t, w2s, b1c, b2s)
    s_flat = s3d.reshape(E_SC)
    flat_idx = idx3d.reshape(E_SC)
    psum, pcnt = _sc_scatter(flat_idx, s_flat)

    return _finish(psum, pcnt, atom_fea, tp_w)


# speedup vs baseline: 4.6814x; 1.0031x over previous
import functools

import jax
import jax.numpy as jnp
from jax import lax
from jax.experimental import pallas as pl
from jax.experimental.pallas import tpu as pltpu
from jax.experimental.pallas import tpu_sc as plsc

N = 10000
M = 32
ATOM = 128
NBR = 16
GROUPS = 8
E = N * M
ROWS = E // GROUPS
NW = 32
NPAD = 10240


BN = 1024
GRID_A = -(-N // BN)
NCOL = GRID_A * BN
E_SC = M * NCOL
TRASH = N


def _edge_scalar_body(fea_ref, idx_ref, w1t_ref, w2s_ref, b1c_ref, b2s_ref,
                      s_ref, oidx_ref):
    x = fea_ref[...].reshape(M * NBR, BN)
    rows = []
    for gp in range(4):
        xg = x[128 * gp:128 * (gp + 1), :]
        z = jnp.dot(w1t_ref[...], xg, preferred_element_type=jnp.float32)
        z = z + b1c_ref[...]
        hg = jnp.maximum(z, 0.0) + jnp.log1p(jnp.exp(-jnp.abs(z)))
        rows.append(jnp.dot(w2s_ref[...], hg,
                            preferred_element_type=jnp.float32))
    s = jnp.concatenate(rows, axis=0) + b2s_ref[0]

    g = pl.program_id(0)
    col = g * BN + lax.broadcasted_iota(jnp.int32, (M, BN), 1)
    idx2 = jnp.where(col < N, idx_ref[...], TRASH)

    s_ref[...] = jnp.stack(
        [s[:, 128 * t:128 * (t + 1)] for t in range(BN // 128)], axis=1)
    oidx_ref[...] = jnp.stack(
        [idx2[:, 128 * t:128 * (t + 1)] for t in range(BN // 128)], axis=1)


def _edge_scalars(fea_t, idx_t, w1t, w2s, b1c, b2s):
    bt = BN // 128
    return pl.pallas_call(
        _edge_scalar_body,
        grid=(GRID_A,),
        in_specs=[
            pl.BlockSpec((M, NBR, BN), lambda g: (0, 0, g)),
            pl.BlockSpec((M, BN), lambda g: (0, g)),
            pl.BlockSpec((128, 128), lambda g: (0, 0)),
            pl.BlockSpec((GROUPS, 128), lambda g: (0, 0)),
            pl.BlockSpec((128, 1), lambda g: (0, 0)),
            pl.BlockSpec(memory_space=pltpu.SMEM),
        ],
        out_specs=[
            pl.BlockSpec((M, bt, 128), lambda g: (0, g, 0)),
            pl.BlockSpec((M, bt, 128), lambda g: (0, g, 0)),
        ],
        out_shape=[
            jax.ShapeDtypeStruct((M, NCOL // 128, 128), jnp.float32),
            jax.ShapeDtypeStruct((M, NCOL // 128, 128), jnp.int32),
        ],
    )(fea_t, idx_t, w1t, w2s, b1c, b2s)


E_PER_W = E_SC // NW
VECS_PER_W = E_PER_W // 16


def _sc_scatter(flat_idx, s_flat):
    mesh = plsc.VectorSubcoreMesh(core_axis_name="c", subcore_axis_name="s")

    @functools.partial(
        pl.kernel,
        mesh=mesh,
        out_type=[
            jax.ShapeDtypeStruct((NW, NPAD), jnp.float32),
            jax.ShapeDtypeStruct((NW, NPAD), jnp.float32),
        ],
        scratch_types=[
            pltpu.VMEM((E_PER_W,), jnp.int32),
            pltpu.VMEM((E_PER_W,), jnp.float32),
            pltpu.VMEM((NPAD,), jnp.float32),
            pltpu.VMEM((NPAD,), jnp.float32),
        ],
        compiler_params=pltpu.CompilerParams(needs_layout_passes=False),
    )
    def scatter_kernel(idx_hbm, s_hbm, osum_hbm, ocnt_hbm,
                       idx_v, s_v, sum_v, cnt_v):
        wid = lax.axis_index("s") * 2 + lax.axis_index("c")
        base = wid * E_PER_W
        pltpu.sync_copy(idx_hbm.at[pl.ds(base, E_PER_W)], idx_v)
        pltpu.sync_copy(s_hbm.at[pl.ds(base, E_PER_W)], s_v)

        zeros = jnp.zeros((16,), jnp.float32)

        def zero_body(i, _):
            for u in range(8):
                off = (i * 8 + u) * 16
                sum_v[pl.ds(off, 16)] = zeros
                cnt_v[pl.ds(off, 16)] = zeros
            return ()

        lax.fori_loop(0, NPAD // 16 // 8, zero_body, ())

        ones = jnp.ones((16,), jnp.float32)

        def acc_body(i, _):
            for u in range(4):
                off = (i * 4 + u) * 16
                idx16 = idx_v[pl.ds(off, 16)]
                s16 = s_v[pl.ds(off, 16)]
                plsc.addupdate_scatter(sum_v, [idx16], s16)
                plsc.addupdate_scatter(cnt_v, [idx16], ones)
            return ()

        lax.fori_loop(0, VECS_PER_W // 4, acc_body, ())

        pltpu.sync_copy(sum_v, osum_hbm.at[wid])
        pltpu.sync_copy(cnt_v, ocnt_hbm.at[wid])

    return scatter_kernel(flat_idx, s_flat)


def _finish_body(sum_ref, cnt_ref, atom_ref, tpw_ref, out_ref):
    ssum = jnp.sum(sum_ref[...], axis=0)
    cnt = jnp.sum(cnt_ref[...], axis=0)
    mean = ssum / jnp.maximum(cnt, 1.0)
    scale = mean * (1.0 / jnp.sqrt(float(ATOM)))
    p = jnp.dot(atom_ref[...], tpw_ref[...], preferred_element_type=jnp.float32)
    out_ref[...] = p * scale[:, None]


def _finish(psum, pcnt, atom_fea, tp_w):
    BC = 1024
    grid = NPAD // BC
    return pl.pallas_call(
        _finish_body,
        grid=(grid,),
        in_specs=[
            pl.BlockSpec((NW, BC), lambda g: (0, g)),
            pl.BlockSpec((NW, BC), lambda g: (0, g)),
            pl.BlockSpec((BC, ATOM), lambda g: (g, 0)),
            pl.BlockSpec((ATOM, ATOM), lambda g: (0, 0)),
        ],
        out_specs=pl.BlockSpec((BC, ATOM), lambda g: (g, 0)),
        out_shape=jax.ShapeDtypeStruct((N, ATOM), jnp.float32),
    )(psum, pcnt, atom_fea, tp_w)


def kernel(atom_fea, nbr_fea, nbr_idx, pos, W1, b1, W2, b2, tp_w):
    del pos
    w1t = jnp.kron(jnp.eye(GROUPS, dtype=jnp.float32), W1.T)
    w2s = jnp.kron(jnp.eye(GROUPS, dtype=jnp.float32),
                   W2[:, 0][None, :])
    b1c = jnp.tile(b1, GROUPS)[:, None]
    b2s = jnp.full((1,), b2[0], dtype=jnp.float32)

    fea_t = jnp.transpose(nbr_fea, (1, 2, 0))
    idx_t = jnp.transpose(nbr_idx.astype(jnp.int32), (1, 0))

    s3d, idx3d = _edge_scalars(fea_t, idx_t, w1t, w2s, b1c, b2s)
    s_flat = s3d.reshape(E_SC)
    flat_idx = idx3d.reshape(E_SC)
    psum, pcnt = _sc_scatter(flat_idx, s_flat)

    return _finish(psum, pcnt, atom_fea, tp_w)


# speedup vs baseline: 5.0355x; 1.0756x over previous
import functools

import jax
import jax.numpy as jnp
from jax import lax
from jax.experimental import pallas as pl
from jax.experimental.pallas import tpu as pltpu
from jax.experimental.pallas import tpu_sc as plsc

N = 10000
M = 32
ATOM = 128
NBR = 16
GROUPS = 8
E = N * M
ROWS = E // GROUPS
NW = 32
NPAD = 10240


BN = 1024
GRID_A = -(-N // BN)
NCOL = GRID_A * BN
E_SC = M * NCOL
TRASH = N


def _edge_scalar_body(fea_ref, idx_ref, w1t_ref, w2s_ref, b1c_ref, b2s_ref,
                      s_ref, oidx_ref):
    x = fea_ref[...].reshape(M * NBR, BN)
    rows = []
    for gp in range(4):
        xg = x[128 * gp:128 * (gp + 1), :]
        z = jnp.dot(w1t_ref[...], xg, preferred_element_type=jnp.float32)
        z = z + b1c_ref[...]
        e = jnp.exp2(jnp.abs(z) * (-1.4426950408889634))
        hg = jnp.maximum(z, 0.0) + 0.6931471805599453 * jnp.log2(1.0 + e)
        rows.append(jnp.dot(w2s_ref[...], hg,
                            preferred_element_type=jnp.float32))
    s = jnp.concatenate(rows, axis=0) + b2s_ref[0]

    g = pl.program_id(0)
    col = g * BN + lax.broadcasted_iota(jnp.int32, (M, BN), 1)
    idx2 = jnp.where(col < N, idx_ref[...], TRASH)

    s_ref[...] = jnp.stack(
        [s[:, 128 * t:128 * (t + 1)] for t in range(BN // 128)], axis=1)
    oidx_ref[...] = jnp.stack(
        [idx2[:, 128 * t:128 * (t + 1)] for t in range(BN // 128)], axis=1)


def _edge_scalars(fea_t, idx_t, w1t, w2s, b1c, b2s):
    bt = BN // 128
    return pl.pallas_call(
        _edge_scalar_body,
        grid=(GRID_A,),
        in_specs=[
            pl.BlockSpec((M, NBR, BN), lambda g: (0, 0, g)),
            pl.BlockSpec((M, BN), lambda g: (0, g)),
            pl.BlockSpec((128, 128), lambda g: (0, 0)),
            pl.BlockSpec((GROUPS, 128), lambda g: (0, 0)),
            pl.BlockSpec((128, 1), lambda g: (0, 0)),
            pl.BlockSpec(memory_space=pltpu.SMEM),
        ],
        out_specs=[
            pl.BlockSpec((M, bt, 128), lambda g: (0, g, 0)),
            pl.BlockSpec((M, bt, 128), lambda g: (0, g, 0)),
        ],
        out_shape=[
            jax.ShapeDtypeStruct((M, NCOL // 128, 128), jnp.float32),
            jax.ShapeDtypeStruct((M, NCOL // 128, 128), jnp.int32),
        ],
    )(fea_t, idx_t, w1t, w2s, b1c, b2s)


E_PER_W = E_SC // NW
VECS_PER_W = E_PER_W // 16


def _sc_scatter(flat_idx, s_flat):
    mesh = plsc.VectorSubcoreMesh(core_axis_name="c", subcore_axis_name="s")

    @functools.partial(
        pl.kernel,
        mesh=mesh,
        out_type=[
            jax.ShapeDtypeStruct((NW, NPAD), jnp.float32),
            jax.ShapeDtypeStruct((NW, NPAD), jnp.float32),
        ],
        scratch_types=[
            pltpu.VMEM((E_PER_W,), jnp.int32),
            pltpu.VMEM((E_PER_W,), jnp.float32),
            pltpu.VMEM((NPAD,), jnp.float32),
            pltpu.VMEM((NPAD,), jnp.float32),
        ],
        compiler_params=pltpu.CompilerParams(needs_layout_passes=False),
    )
    def scatter_kernel(idx_hbm, s_hbm, osum_hbm, ocnt_hbm,
                       idx_v, s_v, sum_v, cnt_v):
        wid = lax.axis_index("s") * 2 + lax.axis_index("c")
        base = wid * E_PER_W
        pltpu.sync_copy(idx_hbm.at[pl.ds(base, E_PER_W)], idx_v)
        pltpu.sync_copy(s_hbm.at[pl.ds(base, E_PER_W)], s_v)

        zeros = jnp.zeros((16,), jnp.float32)

        def zero_body(i, _):
            for u in range(8):
                off = (i * 8 + u) * 16
                sum_v[pl.ds(off, 16)] = zeros
                cnt_v[pl.ds(off, 16)] = zeros
            return ()

        lax.fori_loop(0, NPAD // 16 // 8, zero_body, ())

        ones = jnp.ones((16,), jnp.float32)

        def acc_body(i, _):
            for u in range(4):
                off = (i * 4 + u) * 16
                idx16 = idx_v[pl.ds(off, 16)]
                s16 = s_v[pl.ds(off, 16)]
                plsc.addupdate_scatter(sum_v, [idx16], s16)
                plsc.addupdate_scatter(cnt_v, [idx16], ones)
            return ()

        lax.fori_loop(0, VECS_PER_W // 4, acc_body, ())

        pltpu.sync_copy(sum_v, osum_hbm.at[wid])
        pltpu.sync_copy(cnt_v, ocnt_hbm.at[wid])

    return scatter_kernel(flat_idx, s_flat)


def _finish_body(sum_ref, cnt_ref, atom_ref, tpw_ref, out_ref):
    ssum = jnp.sum(sum_ref[...], axis=0)
    cnt = jnp.sum(cnt_ref[...], axis=0)
    mean = ssum / jnp.maximum(cnt, 1.0)
    scale = mean * (1.0 / jnp.sqrt(float(ATOM)))
    p = jnp.dot(atom_ref[...], tpw_ref[...], preferred_element_type=jnp.float32)
    out_ref[...] = p * scale[:, None]


def _finish(psum, pcnt, atom_fea, tp_w):
    BC = 2048
    grid = NPAD // BC
    return pl.pallas_call(
        _finish_body,
        grid=(grid,),
        in_specs=[
            pl.BlockSpec((NW, BC), lambda g: (0, g)),
            pl.BlockSpec((NW, BC), lambda g: (0, g)),
            pl.BlockSpec((BC, ATOM), lambda g: (g, 0)),
            pl.BlockSpec((ATOM, ATOM), lambda g: (0, 0)),
        ],
        out_specs=pl.BlockSpec((BC, ATOM), lambda g: (g, 0)),
        out_shape=jax.ShapeDtypeStruct((N, ATOM), jnp.float32),
    )(psum, pcnt, atom_fea, tp_w)


def kernel(atom_fea, nbr_fea, nbr_idx, pos, W1, b1, W2, b2, tp_w):
    del pos
    w1t = jnp.kron(jnp.eye(GROUPS, dtype=jnp.float32), W1.T)
    w2s = jnp.kron(jnp.eye(GROUPS, dtype=jnp.float32),
                   W2[:, 0][None, :])
    b1c = jnp.tile(b1, GROUPS)[:, None]
    b2s = jnp.full((1,), b2[0], dtype=jnp.float32)

    fea_t = jnp.transpose(nbr_fea, (1, 2, 0))
    idx_t = jnp.transpose(nbr_idx.astype(jnp.int32), (1, 0))

    s3d, idx3d = _edge_scalars(fea_t, idx_t, w1t, w2s, b1c, b2s)
    s_flat = s3d.reshape(E_SC)
    flat_idx = idx3d.reshape(E_SC)
    psum, pcnt = _sc_scatter(flat_idx, s_flat)

    return _finish(psum, pcnt, atom_fea, tp_w)


# speedup vs baseline: 5.4138x; 1.0751x over previous
import functools

import jax
import jax.numpy as jnp
from jax import lax
from jax.experimental import pallas as pl
from jax.experimental.pallas import tpu as pltpu
from jax.experimental.pallas import tpu_sc as plsc

N = 10000
M = 32
ATOM = 128
NBR = 16
GROUPS = 8
E = N * M
ROWS = E // GROUPS
NW = 32
NPAD = 10240


BN = 2048
GRID_A = -(-N // BN)
NCOL = GRID_A * BN
E_SC = M * NCOL
TRASH = N


def _edge_scalar_body(fea_ref, idx_ref, w1t_ref, w2s_ref, b1c_ref, b2s_ref,
                      s_ref, oidx_ref):
    x = fea_ref[...].reshape(M * NBR, BN)
    rows = []
    for gp in range(4):
        xg = x[128 * gp:128 * (gp + 1), :]
        z = jnp.dot(w1t_ref[...], xg, preferred_element_type=jnp.float32)
        z = z + b1c_ref[...]
        e = jnp.exp2(jnp.abs(z) * (-1.4426950408889634))
        hg = jnp.maximum(z, 0.0) + 0.6931471805599453 * jnp.log2(1.0 + e)
        rows.append(jnp.dot(w2s_ref[...], hg,
                            preferred_element_type=jnp.float32))
    s = jnp.concatenate(rows, axis=0) + b2s_ref[0]

    g = pl.program_id(0)
    col = g * BN + lax.broadcasted_iota(jnp.int32, (M, BN), 1)
    idx2 = jnp.where(col < N, idx_ref[...], TRASH)

    s_ref[...] = jnp.stack(
        [s[:, 128 * t:128 * (t + 1)] for t in range(BN // 128)], axis=1)
    oidx_ref[...] = jnp.stack(
        [idx2[:, 128 * t:128 * (t + 1)] for t in range(BN // 128)], axis=1)


def _edge_scalars(fea_t, idx_t, w1t, w2s, b1c, b2s):
    bt = BN // 128
    return pl.pallas_call(
        _edge_scalar_body,
        grid=(GRID_A,),
        in_specs=[
            pl.BlockSpec((M, NBR, BN), lambda g: (0, 0, g)),
            pl.BlockSpec((M, BN), lambda g: (0, g)),
            pl.BlockSpec((128, 128), lambda g: (0, 0)),
            pl.BlockSpec((GROUPS, 128), lambda g: (0, 0)),
            pl.BlockSpec((128, 1), lambda g: (0, 0)),
            pl.BlockSpec(memory_space=pltpu.SMEM),
        ],
        out_specs=[
            pl.BlockSpec((M, bt, 128), lambda g: (0, g, 0)),
            pl.BlockSpec((M, bt, 128), lambda g: (0, g, 0)),
        ],
        out_shape=[
            jax.ShapeDtypeStruct((M, NCOL // 128, 128), jnp.float32),
            jax.ShapeDtypeStruct((M, NCOL // 128, 128), jnp.int32),
        ],
    )(fea_t, idx_t, w1t, w2s, b1c, b2s)


E_PER_W = E_SC // NW
VECS_PER_W = E_PER_W // 16


def _sc_scatter(flat_idx, s_flat):
    mesh = plsc.VectorSubcoreMesh(core_axis_name="c", subcore_axis_name="s")

    @functools.partial(
        pl.kernel,
        mesh=mesh,
        out_type=[
            jax.ShapeDtypeStruct((NW, NPAD), jnp.float32),
            jax.ShapeDtypeStruct((NW, NPAD), jnp.float32),
        ],
        scratch_types=[
            pltpu.VMEM((E_PER_W,), jnp.int32),
            pltpu.VMEM((E_PER_W,), jnp.float32),
            pltpu.VMEM((NPAD,), jnp.float32),
            pltpu.VMEM((NPAD,), jnp.float32),
        ],
        compiler_params=pltpu.CompilerParams(needs_layout_passes=False),
    )
    def scatter_kernel(idx_hbm, s_hbm, osum_hbm, ocnt_hbm,
                       idx_v, s_v, sum_v, cnt_v):
        wid = lax.axis_index("s") * 2 + lax.axis_index("c")
        base = wid * E_PER_W
        pltpu.sync_copy(idx_hbm.at[pl.ds(base, E_PER_W)], idx_v)
        pltpu.sync_copy(s_hbm.at[pl.ds(base, E_PER_W)], s_v)

        zeros = jnp.zeros((16,), jnp.float32)

        @plsc.parallel_loop(0, NPAD // 16, unroll=8)
        def _(i):
            sum_v[pl.ds(i * 16, 16)] = zeros
            cnt_v[pl.ds(i * 16, 16)] = zeros

        ones = jnp.ones((16,), jnp.float32)

        @plsc.parallel_loop(0, VECS_PER_W, unroll=8)
        def _(i):
            idx16 = idx_v[pl.ds(i * 16, 16)]
            s16 = s_v[pl.ds(i * 16, 16)]
            plsc.addupdate_scatter(sum_v, [idx16], s16)
            plsc.addupdate_scatter(cnt_v, [idx16], ones)

        pltpu.sync_copy(sum_v, osum_hbm.at[wid])
        pltpu.sync_copy(cnt_v, ocnt_hbm.at[wid])

    return scatter_kernel(flat_idx, s_flat)


def _finish_body(sum_ref, cnt_ref, atom_ref, tpw_ref, out_ref):
    ssum = jnp.sum(sum_ref[...], axis=0)
    cnt = jnp.sum(cnt_ref[...], axis=0)
    mean = ssum / jnp.maximum(cnt, 1.0)
    scale = mean * (1.0 / jnp.sqrt(float(ATOM)))
    p = jnp.dot(atom_ref[...], tpw_ref[...], preferred_element_type=jnp.float32)
    out_ref[...] = p * scale[:, None]


def _finish(psum, pcnt, atom_fea, tp_w):
    BC = 2048
    grid = NPAD // BC
    return pl.pallas_call(
        _finish_body,
        grid=(grid,),
        in_specs=[
            pl.BlockSpec((NW, BC), lambda g: (0, g)),
            pl.BlockSpec((NW, BC), lambda g: (0, g)),
            pl.BlockSpec((BC, ATOM), lambda g: (g, 0)),
            pl.BlockSpec((ATOM, ATOM), lambda g: (0, 0)),
        ],
        out_specs=pl.BlockSpec((BC, ATOM), lambda g: (g, 0)),
        out_shape=jax.ShapeDtypeStruct((N, ATOM), jnp.float32),
    )(psum, pcnt, atom_fea, tp_w)


def kernel(atom_fea, nbr_fea, nbr_idx, pos, W1, b1, W2, b2, tp_w):
    del pos
    w1t = jnp.kron(jnp.eye(GROUPS, dtype=jnp.float32), W1.T)
    w2s = jnp.kron(jnp.eye(GROUPS, dtype=jnp.float32),
                   W2[:, 0][None, :])
    b1c = jnp.tile(b1, GROUPS)[:, None]
    b2s = jnp.full((1,), b2[0], dtype=jnp.float32)

    fea_t = jnp.transpose(nbr_fea, (1, 2, 0))
    idx_t = jnp.transpose(nbr_idx.astype(jnp.int32), (1, 0))

    s3d, idx3d = _edge_scalars(fea_t, idx_t, w1t, w2s, b1c, b2s)
    s_flat = s3d.reshape(E_SC)
    flat_idx = idx3d.reshape(E_SC)
    psum, pcnt = _sc_scatter(flat_idx, s_flat)

    return _finish(psum, pcnt, atom_fea, tp_w)


# speedup vs baseline: 5.8757x; 1.0853x over previous
import functools

import jax
import jax.numpy as jnp
from jax import lax
from jax.experimental import pallas as pl
from jax.experimental.pallas import tpu as pltpu
from jax.experimental.pallas import tpu_sc as plsc

N = 10000
M = 32
ATOM = 128
NBR = 16
GROUPS = 8
E = N * M
ROWS = E // GROUPS
NW = 32
NPAD = 10240


BN = 2048
GRID_A = -(-N // BN)
NCOL = GRID_A * BN
E_SC = M * NCOL
TRASH = N


def _edge_scalar_body(fea_ref, idx_ref, w1_ref, w2_ref, b1c_ref, b2_ref,
                      s_ref, oidx_ref):
    rr = lax.broadcasted_iota(jnp.int32, (128, 128), 0) >> 4
    cc = lax.broadcasted_iota(jnp.int32, (128, 128), 1) >> 4
    w1T = jnp.transpose(w1_ref[...], (1, 0))
    row16 = jnp.concatenate([w1T] * 8, axis=1)
    w1t = jnp.concatenate([row16] * 8, axis=0)
    w1t = jnp.where(rr == cc, w1t, 0.0)
    w2cT = jnp.transpose(w2_ref[...], (1, 0))
    w2row = jnp.concatenate([w2cT] * 8, axis=1)
    w2s = jnp.broadcast_to(w2row, (GROUPS, 128))
    rr8 = lax.broadcasted_iota(jnp.int32, (GROUPS, 128), 0)
    cc8 = lax.broadcasted_iota(jnp.int32, (GROUPS, 128), 1) >> 4
    w2s = jnp.where(rr8 == cc8, w2s, 0.0)

    x = fea_ref[...].reshape(M * NBR, BN)
    rows = []
    for gp in range(4):
        xg = x[128 * gp:128 * (gp + 1), :]
        z = jnp.dot(w1t, xg, preferred_element_type=jnp.float32)
        z = z + b1c_ref[...]
        e = jnp.exp2(jnp.abs(z) * (-1.4426950408889634))
        hg = jnp.maximum(z, 0.0) + 0.6931471805599453 * jnp.log2(1.0 + e)
        rows.append(jnp.dot(w2s, hg, preferred_element_type=jnp.float32))
    s = jnp.concatenate(rows, axis=0) + b2_ref[0]

    g = pl.program_id(0)
    col = g * BN + lax.broadcasted_iota(jnp.int32, (M, BN), 1)
    idx2 = jnp.where(col < N, idx_ref[...], TRASH)

    s_ref[...] = jnp.stack(
        [s[:, 128 * t:128 * (t + 1)] for t in range(BN // 128)], axis=1)
    oidx_ref[...] = jnp.stack(
        [idx2[:, 128 * t:128 * (t + 1)] for t in range(BN // 128)], axis=1)


def _edge_scalars(fea_t, idx_t, w1, w2col, b1c, b2):
    bt = BN // 128
    return pl.pallas_call(
        _edge_scalar_body,
        grid=(GRID_A,),
        in_specs=[
            pl.BlockSpec((M, NBR, BN), lambda g: (0, 0, g)),
            pl.BlockSpec((M, BN), lambda g: (0, g)),
            pl.BlockSpec((NBR, NBR), lambda g: (0, 0)),
            pl.BlockSpec((NBR, 1), lambda g: (0, 0)),
            pl.BlockSpec((128, 1), lambda g: (0, 0)),
            pl.BlockSpec(memory_space=pltpu.SMEM),
        ],
        out_specs=[
            pl.BlockSpec((M, bt, 128), lambda g: (0, g, 0)),
            pl.BlockSpec((M, bt, 128), lambda g: (0, g, 0)),
        ],
        out_shape=[
            jax.ShapeDtypeStruct((M, NCOL // 128, 128), jnp.float32),
            jax.ShapeDtypeStruct((M, NCOL // 128, 128), jnp.int32),
        ],
    )(fea_t, idx_t, w1, w2col, b1c, b2)


E_PER_W = E_SC // NW
VECS_PER_W = E_PER_W // 16


def _sc_scatter(flat_idx, s_flat):
    mesh = plsc.VectorSubcoreMesh(core_axis_name="c", subcore_axis_name="s")

    @functools.partial(
        pl.kernel,
        mesh=mesh,
        out_type=[
            jax.ShapeDtypeStruct((NW, NPAD), jnp.float32),
            jax.ShapeDtypeStruct((NW, NPAD), jnp.float32),
        ],
        scratch_types=[
            pltpu.VMEM((E_PER_W,), jnp.int32),
            pltpu.VMEM((E_PER_W,), jnp.float32),
            pltpu.VMEM((NPAD,), jnp.float32),
            pltpu.VMEM((NPAD,), jnp.float32),
        ],
        compiler_params=pltpu.CompilerParams(needs_layout_passes=False),
    )
    def scatter_kernel(idx_hbm, s_hbm, osum_hbm, ocnt_hbm,
                       idx_v, s_v, sum_v, cnt_v):
        wid = lax.axis_index("s") * 2 + lax.axis_index("c")
        base = wid * E_PER_W
        pltpu.sync_copy(idx_hbm.at[pl.ds(base, E_PER_W)], idx_v)
        pltpu.sync_copy(s_hbm.at[pl.ds(base, E_PER_W)], s_v)

        zeros = jnp.zeros((16,), jnp.float32)

        @plsc.parallel_loop(0, NPAD // 16, unroll=8)
        def _(i):
            sum_v[pl.ds(i * 16, 16)] = zeros
            cnt_v[pl.ds(i * 16, 16)] = zeros

        ones = jnp.ones((16,), jnp.float32)

        @plsc.parallel_loop(0, VECS_PER_W, unroll=8)
        def _(i):
            idx16 = idx_v[pl.ds(i * 16, 16)]
            s16 = s_v[pl.ds(i * 16, 16)]
            plsc.addupdate_scatter(sum_v, [idx16], s16)
            plsc.addupdate_scatter(cnt_v, [idx16], ones)

        pltpu.sync_copy(sum_v, osum_hbm.at[wid])
        pltpu.sync_copy(cnt_v, ocnt_hbm.at[wid])

    return scatter_kernel(flat_idx, s_flat)


def _finish_body(sum_ref, cnt_ref, atom_ref, tpw_ref, out_ref):
    ssum = jnp.sum(sum_ref[...], axis=0)
    cnt = jnp.sum(cnt_ref[...], axis=0)
    mean = ssum / jnp.maximum(cnt, 1.0)
    scale = mean * (1.0 / jnp.sqrt(float(ATOM)))
    p = jnp.dot(atom_ref[...], tpw_ref[...], preferred_element_type=jnp.float32)
    out_ref[...] = p * scale[:, None]


def _finish(psum, pcnt, atom_fea, tp_w):
    BC = 2048
    grid = NPAD // BC
    return pl.pallas_call(
        _finish_body,
        grid=(grid,),
        in_specs=[
            pl.BlockSpec((NW, BC), lambda g: (0, g)),
            pl.BlockSpec((NW, BC), lambda g: (0, g)),
            pl.BlockSpec((BC, ATOM), lambda g: (g, 0)),
            pl.BlockSpec((ATOM, ATOM), lambda g: (0, 0)),
        ],
        out_specs=pl.BlockSpec((BC, ATOM), lambda g: (g, 0)),
        out_shape=jax.ShapeDtypeStruct((N, ATOM), jnp.float32),
    )(psum, pcnt, atom_fea, tp_w)


def kernel(atom_fea, nbr_fea, nbr_idx, pos, W1, b1, W2, b2, tp_w):
    del pos
    b1c = jnp.tile(b1, GROUPS)[:, None]

    fea_t = jnp.transpose(nbr_fea, (1, 2, 0))
    idx_t = jnp.transpose(nbr_idx.astype(jnp.int32), (1, 0))

    s3d, idx3d = _edge_scalars(fea_t, idx_t, W1, W2[:, 0:1], b1c, b2)
    s_flat = s3d.reshape(E_SC)
    flat_idx = idx3d.reshape(E_SC)
    psum, pcnt = _sc_scatter(flat_idx, s_flat)

    return _finish(psum, pcnt, atom_fea, tp_w)
